# baseline (speedup 1.0000x reference)
import functools

import jax
import jax.numpy as jnp
from jax import lax
from jax.experimental import pallas as pl
from jax.experimental.pallas import tpu as pltpu
from jax.experimental.pallas import tpu_sc as plsc

_N, _C, _J = 65536, 80, 600
_NW = 32
_RPW = _N // _NW
_RB = 64
_NCH = _RPW // _RB
_NG = _J // 16
_JP = 608

_mesh = plsc.VectorSubcoreMesh(core_axis_name="c", subcore_axis_name="s")


def _sc_body(x_hbm, idx_hbm, out_hbm, idx_v, in2_v, in_v, out_v):
    cid = lax.axis_index("c")
    sid = lax.axis_index("s")
    wid = sid * 2 + cid
    row0 = wid * _RPW

    idx_v[pl.ds(592, 16)] = jnp.zeros((16,), jnp.int32)
    pltpu.sync_copy(idx_hbm, idx_v.at[pl.ds(0, _J)])

    iota = lax.iota(jnp.int32, 16)
    ones = jnp.ones((16,), jnp.int32)
    tail_mask = iota < (_J - _NG * 16)
    tail_j = iota + (_NG * 16)

    def column_pass(g):
        src0 = idx_v[pl.ds(g * 16, 16)]

        @plsc.parallel_loop(0, _RB, 1, unroll=8, carry=src0)
        def _(r, src):
            v = plsc.load_gather(in_v, [src])
            out_v[r, pl.ds(g * 16, 16)] = v
            return src + _C

    def tail_pass():
        src0 = idx_v[pl.ds(_NG * 16, 16)]

        @plsc.parallel_loop(0, _RB, 1, unroll=4, carry=(src0, jnp.zeros((16,), jnp.int32)))
        def _(r, c):
            src, r_vec = c
            v = plsc.load_gather(in_v, [src], mask=tail_mask)
            plsc.store_scatter(out_v, [r_vec, tail_j], v, mask=tail_mask)
            return (src + _C, r_vec + ones)

    def chunk_body(k, _):
        r0 = row0 + k * _RB
        pltpu.sync_copy(x_hbm.at[pl.ds(r0, _RB)], in2_v)

        @plsc.parallel_loop(0, _RB, 1, unroll=2)
        def _(r):
            for t in range(_C // 16):
                in_v[pl.ds(r * _C + t * 16, 16)] = in2_v[r, pl.ds(t * 16, 16)]

        for g in range(_NG):
            column_pass(g)
        tail_pass()
        pltpu.sync_copy(out_v, out_hbm.at[pl.ds(r0, _RB)])
        return 0

    lax.fori_loop(0, _NCH, chunk_body, 0)


_sc_call = functools.partial(
    pl.kernel,
    out_type=jax.ShapeDtypeStruct((_N, _J), jnp.float32),
    mesh=_mesh,
    compiler_params=pltpu.CompilerParams(
        needs_layout_passes=False, disable_bounds_checks=True,
        use_tc_tiling_on_sc=True),
    scratch_types=[
        pltpu.VMEM((_JP,), jnp.int32),
        pltpu.VMEM((_RB, _C), jnp.float32),
        pltpu.VMEM((_RB * _C,), jnp.float32),
        pltpu.VMEM((_RB, _J), jnp.float32),
    ],
)(_sc_body)


def kernel(o_conf, obj2hoi):
    return _sc_call(o_conf, obj2hoi.astype(jnp.int32))


# speedup vs baseline: 1.0804x; 1.0804x over previous
import functools

import jax
import jax.numpy as jnp
from jax import lax
from jax.experimental import pallas as pl
from jax.experimental.pallas import tpu as pltpu
from jax.experimental.pallas import tpu_sc as plsc

_N, _C, _J = 65536, 80, 600
_NW = 32
_RPW = _N // _NW
_RB = 64
_NCH = _RPW // _RB
_NG = _J // 16
_JP = 608

_mesh = plsc.VectorSubcoreMesh(core_axis_name="c", subcore_axis_name="s")


def _sc_body(x_hbm, idx_hbm, out_hbm, idx_v, in2_v, in_v, out0_v, out1_v,
             sem0, sem1):
    cid = lax.axis_index("c")
    sid = lax.axis_index("s")
    wid = sid * 2 + cid
    row0 = wid * _RPW

    idx_v[pl.ds(592, 16)] = jnp.zeros((16,), jnp.int32)
    pltpu.sync_copy(idx_hbm, idx_v.at[pl.ds(0, _J)])

    iota = lax.iota(jnp.int32, 16)
    ones = jnp.ones((16,), jnp.int32)
    tail_mask = iota < (_J - _NG * 16)
    tail_j = iota + (_NG * 16)

    def gather_chunk(k, out_v):
        r0 = row0 + k * _RB
        pltpu.sync_copy(x_hbm.at[pl.ds(r0, _RB)], in2_v)

        @plsc.parallel_loop(0, _RB, 1, unroll=2)
        def _(r):
            for t in range(_C // 16):
                in_v[pl.ds(r * _C + t * 16, 16)] = in2_v[r, pl.ds(t * 16, 16)]

        for g in range(_NG):
            src0 = idx_v[pl.ds(g * 16, 16)]

            @plsc.parallel_loop(0, _RB, 1, unroll=8, carry=src0)
            def _(r, src):
                v = plsc.load_gather(in_v, [src])
                out_v[r, pl.ds(g * 16, 16)] = v
                return src + _C

        src0 = idx_v[pl.ds(_NG * 16, 16)]

        @plsc.parallel_loop(0, _RB, 1, unroll=4,
                            carry=(src0, jnp.zeros((16,), jnp.int32)))
        def _(r, c):
            src, r_vec = c
            v = plsc.load_gather(in_v, [src], mask=tail_mask)
            plsc.store_scatter(out_v, [r_vec, tail_j], v, mask=tail_mask)
            return (src + _C, r_vec + ones)

    def out_copy(k, out_v, sem):
        r0 = row0 + k * _RB
        return pltpu.make_async_copy(out_v, out_hbm.at[pl.ds(r0, _RB)], sem)

    bufs = ((out0_v, sem0), (out1_v, sem1))

    for b in (0, 1):
        gather_chunk(b, bufs[b][0])
        out_copy(b, *bufs[b]).start()

    def loop_body(k2, _):
        k = 2 * k2
        for b in (0, 1):
            out_copy(k + b, *bufs[b]).wait()
            gather_chunk(k + b, bufs[b][0])
            out_copy(k + b, *bufs[b]).start()
        return 0

    lax.fori_loop(1, _NCH // 2, loop_body, 0)
    for b in (0, 1):
        out_copy(_NCH - 2 + b, *bufs[b]).wait()


_sc_call = functools.partial(
    pl.kernel,
    out_type=jax.ShapeDtypeStruct((_N, _J), jnp.float32),
    mesh=_mesh,
    compiler_params=pltpu.CompilerParams(
        needs_layout_passes=False, disable_bounds_checks=True),
    scratch_types=[
        pltpu.VMEM((_JP,), jnp.int32),
        pltpu.VMEM((_RB, _C), jnp.float32),
        pltpu.VMEM((_RB * _C,), jnp.float32),
        pltpu.VMEM((_RB, _J), jnp.float32),
        pltpu.VMEM((_RB, _J), jnp.float32),
        pltpu.SemaphoreType.DMA,
        pltpu.SemaphoreType.DMA,
    ],
)(_sc_body)


def kernel(o_conf, obj2hoi):
    return _sc_call(o_conf, obj2hoi.astype(jnp.int32))


# speedup vs baseline: 1.2198x; 1.1291x over previous
import functools

import jax
import jax.numpy as jnp
from jax import lax
from jax.experimental import pallas as pl
from jax.experimental.pallas import tpu as pltpu
from jax.experimental.pallas import tpu_sc as plsc

_N, _C, _J = 65536, 80, 600
_NW = 32
_RPW = _N // _NW
_RB = 64
_NCH = _RPW // _RB
_NG = _J // 16
_JP = 608

_mesh = plsc.VectorSubcoreMesh(core_axis_name="c", subcore_axis_name="s")


def _sc_body(x_hbm, idx_hbm, out_hbm, idx_v, in2a_v, in2b_v, in_v,
             out0_v, out1_v, isem0, isem1, osem0, osem1):
    cid = lax.axis_index("c")
    sid = lax.axis_index("s")
    wid = sid * 2 + cid
    row0 = wid * _RPW

    idx_v[pl.ds(592, 16)] = jnp.zeros((16,), jnp.int32)
    pltpu.sync_copy(idx_hbm, idx_v.at[pl.ds(0, _J)])

    iota = lax.iota(jnp.int32, 16)
    ones = jnp.ones((16,), jnp.int32)
    tail_mask = iota < (_J - _NG * 16)
    tail_j = iota + (_NG * 16)

    ins = ((in2a_v, isem0), (in2b_v, isem1))
    outs = ((out0_v, osem0), (out1_v, osem1))

    def in_copy(k, in2_v, sem):
        r0 = row0 + k * _RB
        return pltpu.make_async_copy(x_hbm.at[pl.ds(r0, _RB)], in2_v, sem)

    def out_copy(k, out_v, sem):
        r0 = row0 + k * _RB
        return pltpu.make_async_copy(out_v, out_hbm.at[pl.ds(r0, _RB)], sem)

    def gather_chunk(k, b):
        in2_v, isem = ins[b]
        out_v = outs[b][0]
        in_copy(k, in2_v, isem).wait()

        @plsc.parallel_loop(0, _RB, 1, unroll=2)
        def _(r):
            for t in range(_C // 16):
                in_v[pl.ds(r * _C + t * 16, 16)] = in2_v[r, pl.ds(t * 16, 16)]

        in_copy(jnp.minimum(k + 2, _NCH - 1), in2_v, isem).start()

        for g in range(_NG):
            src0 = idx_v[pl.ds(g * 16, 16)]

            @plsc.parallel_loop(0, _RB, 1, unroll=8, carry=src0)
            def _(r, src):
                v = plsc.load_gather(in_v, [src])
                out_v[r, pl.ds(g * 16, 16)] = v
                return src + _C

        src0 = idx_v[pl.ds(_NG * 16, 16)]

        @plsc.parallel_loop(0, _RB, 1, unroll=4,
                            carry=(src0, jnp.zeros((16,), jnp.int32)))
        def _(r, c):
            src, r_vec = c
            v = plsc.load_gather(in_v, [src], mask=tail_mask)
            plsc.store_scatter(out_v, [r_vec, tail_j], v, mask=tail_mask)
            return (src + _C, r_vec + ones)

    for b in (0, 1):
        in_copy(b, *ins[b]).start()
    for b in (0, 1):
        gather_chunk(b, b)
        out_copy(b, *outs[b]).start()

    def loop_body(k2, _):
        k = 2 * k2
        for b in (0, 1):
            out_copy(k + b, *outs[b]).wait()
            gather_chunk(k + b, b)
            out_copy(k + b, *outs[b]).start()
        return 0

    lax.fori_loop(1, _NCH // 2, loop_body, 0)
    for b in (0, 1):
        out_copy(_NCH - 2 + b, *outs[b]).wait()
        in_copy(_NCH - 1, ins[b][0], ins[b][1]).wait()


_sc_call = functools.partial(
    pl.kernel,
    out_type=jax.ShapeDtypeStruct((_N, _J), jnp.float32),
    mesh=_mesh,
    compiler_params=pltpu.CompilerParams(
        needs_layout_passes=False, disable_bounds_checks=True),
    scratch_types=[
        pltpu.VMEM((_JP,), jnp.int32),
        pltpu.VMEM((_RB, _C), jnp.float32),
        pltpu.VMEM((_RB, _C), jnp.float32),
        pltpu.VMEM((_RB * _C,), jnp.float32),
        pltpu.VMEM((_RB, _J), jnp.float32),
        pltpu.VMEM((_RB, _J), jnp.float32),
        pltpu.SemaphoreType.DMA,
        pltpu.SemaphoreType.DMA,
        pltpu.SemaphoreType.DMA,
        pltpu.SemaphoreType.DMA,
    ],
)(_sc_body)


def kernel(o_conf, obj2hoi):
    return _sc_call(o_conf, obj2hoi.astype(jnp.int32))
